# baseline (speedup 1.0000x reference)
import functools
from math import pi as PI

import jax
import jax.numpy as jnp
import numpy as np
from jax import lax
from jax.experimental import pallas as pl
from jax.experimental.pallas import tpu as pltpu
from jax.experimental.pallas import tpu_sc as plsc

N_NODES = 10000
N_EDGES = 160000
SCA = 128
VEC = 32
ECH = 64
NET = 4
CUTOFF = 10.0
NG = ECH - NET
EPS = 1e-6

NODE_BLK = 2048
NODE_PAD = 10240
EDGE_BLK = 4096
EDGE_PAD = 163840

_F32 = jnp.float32

NC = 2
NS = 16
NW = NC * NS
CAT = SCA + 3 * VEC
GCAT = 256
HALF = 128
GCH = 256
GPW = EDGE_PAD // NW
SCH = 256
SPT = EDGE_PAD // NS


def _sc_gather(table, col2):
    mesh = plsc.VectorSubcoreMesh(core_axis_name="c", subcore_axis_name="s")

    @functools.partial(
        pl.kernel, mesh=mesh,
        out_type=jax.ShapeDtypeStruct((EDGE_PAD, GCAT), _F32),
        scratch_types=[pltpu.VMEM((8, 128), jnp.int32),
                       pltpu.VMEM((GCH, GCAT), _F32),
                       pltpu.SemaphoreType.DMA],
    )
    def k(table_h, col_h, out_h, idx_v, rows_v, sem):
        w = lax.axis_index("s") * NC + lax.axis_index("c")
        for t in range(GPW // 1024):
            r = pl.multiple_of((t * NW + w) * 8, 8)
            pltpu.sync_copy(col_h.at[pl.ds(r, 8)], idx_v)
            for u in range(1024 // GCH):
                cps = [pltpu.async_copy(
                    table_h.at[idx_v.at[u * (GCH // 128) + j]],
                    rows_v.at[pl.ds(j * 128, 128)], sem)
                    for j in range(GCH // 128)]
                for cp in cps:
                    cp.wait()
                pltpu.sync_copy(
                    rows_v,
                    out_h.at[pl.ds((t * NW + w) * 1024 + u * GCH, GCH)])

    return k(table, col2)


def _sc_scatter(row2, msgL, msgR, zeros_h):
    mesh = plsc.VectorSubcoreMesh(core_axis_name="c", subcore_axis_name="s")

    @functools.partial(
        pl.kernel, mesh=mesh,
        out_type=[jax.ShapeDtypeStruct((NODE_PAD, HALF), _F32)] * 2,
        scratch_types=[pltpu.VMEM((8, 128), jnp.int32),
                       pltpu.VMEM((SCH, HALF), _F32),
                       pltpu.VMEM_SHARED((NODE_PAD, HALF), _F32)],
    )
    def k(row_h, msgL_h, msgR_h, z_h, outL_h, outR_h, idx_v, buf_v, acc_s):
        ci = lax.axis_index("c")
        si = lax.axis_index("s")
        zrows = NODE_PAD // NS
        z0 = pl.multiple_of(si * zrows, 8)
        pltpu.sync_copy(z_h, acc_s.at[pl.ds(z0, zrows)])
        plsc.subcore_barrier()
        for t in range(SPT // 1024):
            r = pl.multiple_of((t * NS + si) * 8, 8)
            pltpu.sync_copy(row_h.at[pl.ds(r, 8)], idx_v)
            for u in range(1024 // SCH):
                base = (t * NS + si) * 1024 + u * SCH

                @pl.when(ci == 0)
                def _():
                    pltpu.sync_copy(msgL_h.at[pl.ds(base, SCH)], buf_v)

                @pl.when(ci == 1)
                def _():
                    pltpu.sync_copy(msgR_h.at[pl.ds(base, SCH)], buf_v)

                for j in range(SCH // 128):
                    pltpu.sync_copy(
                        buf_v.at[pl.ds(j * 128, 128)],
                        acc_s.at[idx_v.at[u * (SCH // 128) + j]], add=True)
        plsc.subcore_barrier()
        orows = NODE_PAD // NS
        o0 = pl.multiple_of(si * orows, 8)

        @pl.when(ci == 0)
        def _():
            pltpu.sync_copy(acc_s.at[pl.ds(o0, orows)],
                            outL_h.at[pl.ds(o0, orows)])

        @pl.when(ci == 1)
        def _():
            pltpu.sync_copy(acc_s.at[pl.ds(o0, orows)],
                            outR_h.at[pl.ds(o0, orows)])

    return k(row2, msgL, msgR, zeros_h)


def _leaky01(x):
    return jnp.where(x >= 0, x, 0.01 * x)


def _dot(a, b):
    return jax.lax.dot_general(a, b, (((1,), (1,)), ((), ())),
                               preferred_element_type=_F32)


def _gvl_full(sca, v0, v1, v2, Wv1, Wv2, Ws1, Ws2, Wg, bg):
    i0 = _dot(v0, Wv1)
    i1 = _dot(v1, Wv1)
    i2 = _dot(v2, Wv1)
    vn = jnp.sqrt(i0 * i0 + i1 * i1 + i2 * i2)
    os_ = _dot(vn, Ws1) + _dot(sca, Ws2)
    o0 = _dot(i0, Wv2)
    o1 = _dot(i1, Wv2)
    o2 = _dot(i2, Wv2)
    g = jax.nn.sigmoid(_dot(os_, Wg) + bg)
    return os_, g * o0, g * o1, g * o2


def _node_pre_body(xs_ref, xv_ref,
                   nWv1, nWv2, nWs1, nWs2, nWg, nbg,
                   cWv1, cWv2, cWs1, cWs2, cWg, cbg,
                   ncat_ref, ccat_ref):
    xs = xs_ref[...]
    xv = xv_ref[...]
    v0 = xv[:, 0 * VEC:1 * VEC]
    v1 = xv[:, 1 * VEC:2 * VEC]
    v2 = xv[:, 2 * VEC:3 * VEC]
    ns, n0, n1, n2 = _gvl_full(xs, v0, v1, v2, nWv1[...], nWv2[...],
                               nWs1[...], nWs2[...], nWg[...], nbg[...])
    cs, c0, c1, c2 = _gvl_full(xs, v0, v1, v2, cWv1[...], cWv2[...],
                               cWs1[...], cWs2[...], cWg[...], cbg[...])
    B = xs.shape[0]
    ncat_ref[...] = jnp.concatenate(
        [ns, n0, n1, n2, jnp.zeros((B, GCAT - CAT), _F32)], axis=1)
    ccat_ref[...] = jnp.concatenate([cs, c0, c1, c2], axis=1)


def _edge_body(eb_ref, ebt_ref, nc_ref,
               s1, Wse2, Wge, bge, w2e, Wd2,
               scaW, scab, e2nW, e2nb, n2eW, n2eb, evnlW,
               mWv1, mWv2, mWs1, mWs2, mWg, mbg,
               msgL_ref, msgR_ref):
    eb = eb_ref[...]
    B = eb.shape[0]
    ebt = ebt_ref[...]
    e0t = ebt[0:1, :]
    e1t = ebt[1:2, :]
    e2t = ebt[2:3, :]
    distt = jnp.sqrt(e0t * e0t + e1t * e1t + e2t * e2t)
    invt = 1.0 / (distt + 1e-7)
    unt = distt * invt
    Cct = 0.5 * (jnp.cos(distt * (PI / CUTOFF)) + 1.0)
    Cct = Cct * (distt <= CUTOFF).astype(_F32)

    def tb(x):
        return x.reshape(B, 1)

    dist = tb(distt)
    un = tb(unt)
    un2 = tb(unt * unt)
    u0 = tb(e0t * invt)
    u1 = tb(e1t * invt)
    u2 = tb(e2t * invt)
    Cc = tb(Cct)

    colf = jax.lax.broadcasted_iota(jnp.int32, (B, ECH), 1)
    step = CUTOFF / (NG - 1)
    off = colf.astype(_F32) * step
    coeff = -0.5 / (step * step)
    gs = jnp.exp(coeff * (dist - off) ** 2)
    feat = jnp.where(colf < NG, gs, 0.0)
    for j in range(NET):
        feat = jnp.where(colf == NG + j, eb[:, 3 + j:4 + j], feat)

    ose = un * s1[...] + _dot(feat, Wse2[...])
    ge = jax.nn.sigmoid(_dot(ose, Wge[...]) + bge[...])
    a = ge * w2e[...]
    h = _dot(ge, Wd2[...])
    dvd = a * h * un2
    dsq = h * h * un2
    mask = (dvd >= 0).astype(_F32)
    ecoef = 0.2 * a + 0.8 * (mask * a + (1.0 - mask) * (a - dvd / (dsq + EPS) * h))
    esca = _leaky01(ose)

    nc = nc_ref[...]
    nsj = nc[:, :SCA]
    nv0 = nc[:, SCA + 0 * VEC:SCA + 1 * VEC]
    nv1 = nc[:, SCA + 1 * VEC:SCA + 2 * VEC]
    nv2 = nc[:, SCA + 2 * VEC:SCA + 3 * VEC]

    ysca = nsj * (_dot(esca, scaW[...]) + scab[...])
    A = _dot(esca, e2nW[...]) + e2nb[...]
    Bc = _dot(nsj, n2eW[...]) + n2eb[...]
    q = _dot(ecoef, evnlW[...])
    Bq = Bc * q
    yv0 = A * nv0 + Bq * u0
    yv1 = A * nv1 + Bq * u1
    yv2 = A * nv2 + Bq * u2

    i0 = _dot(yv0, mWv1[...])
    i1 = _dot(yv1, mWv1[...])
    i2 = _dot(yv2, mWv1[...])
    vn = jnp.sqrt(i0 * i0 + i1 * i1 + i2 * i2)
    osm = _dot(vn, mWs1[...]) + _dot(ysca, mWs2[...])
    gm = jax.nn.sigmoid(_dot(osm, mWg[...]) + mbg[...])
    o0 = gm * _dot(i0, mWv2[...])
    o1 = gm * _dot(i1, mWv2[...])
    o2 = gm * _dot(i2, mWv2[...])

    msgL_ref[...] = osm * Cc
    msgR_ref[...] = jnp.concatenate(
        [o0 * Cc, o1 * Cc, o2 * Cc,
         jnp.zeros((B, HALF - 3 * VEC), _F32)], axis=1)


def _node_out_body(aggL_ref, aggR_ref, ccat_ref,
                   lnsw, lnsb, lnvw, lnvb, Wdv,
                   oWv1, oWv2, oWs1, oWs2, oWg, obg,
                   os_ref, ov_ref):
    agL = aggL_ref[...]
    agR = aggR_ref[...]
    cc = ccat_ref[...]
    s = agL + cc[:, :SCA]
    vall = agR[:, :3 * VEC] + cc[:, SCA:]

    m = jnp.mean(s, axis=1, keepdims=True)
    var = jnp.mean((s - m) ** 2, axis=1, keepdims=True)
    sN = (s - m) / jnp.sqrt(var + 1e-5) * lnsw[...] + lnsb[...]

    m2 = jnp.mean(vall, axis=1, keepdims=True)
    var2 = jnp.mean((vall - m2) ** 2, axis=1, keepdims=True)
    vN = (vall - m2) / jnp.sqrt(var2 + 1e-5) * lnvw[...] + lnvb[...]
    v0 = vN[:, 0 * VEC:1 * VEC]
    v1 = vN[:, 1 * VEC:2 * VEC]
    v2 = vN[:, 2 * VEC:3 * VEC]

    sA = _leaky01(sN)
    d0 = _dot(v0, Wdv[...])
    d1 = _dot(v1, Wdv[...])
    d2 = _dot(v2, Wdv[...])
    dvd = v0 * d0 + v1 * d1 + v2 * d2
    dsq = d0 * d0 + d1 * d1 + d2 * d2
    mask = (dvd >= 0).astype(_F32)
    co = dvd / (dsq + EPS)
    w0 = 0.2 * v0 + 0.8 * (mask * v0 + (1.0 - mask) * (v0 - co * d0))
    w1 = 0.2 * v1 + 0.8 * (mask * v1 + (1.0 - mask) * (v1 - co * d1))
    w2 = 0.2 * v2 + 0.8 * (mask * v2 + (1.0 - mask) * (v2 - co * d2))

    fs, f0, f1, f2 = _gvl_full(sA, w0, w1, w2, oWv1[...], oWv2[...],
                               oWs1[...], oWs2[...], oWg[...], obg[...])
    os_ref[...] = fs
    ov_ref[...] = jnp.concatenate([f0, f1, f2], axis=1)


def _full_spec(shape):
    nd = len(shape)
    return pl.BlockSpec(shape, lambda i, _n=nd: (0,) * _n)


def _row_spec(blk, ncols):
    return pl.BlockSpec((blk, ncols), lambda i: (i, 0))


def kernel(x_scalar, x_vector, edge_feature, edge_vector, params, edge_index):
    p = params
    f32 = _F32

    def r1(x):
        return x.reshape(1, -1).astype(f32)

    ngvl, cgvl, egvl, mgvl, ogvl = (p['node_gvl'], p['centroid_gvl'],
                                    p['edge_gvl'], p['mm_out_gvl'],
                                    p['out_gvl'])
    w1e = egvl['Wv1'] @ p['ee_w']
    w2e = egvl['Wv2'] @ w1e
    s1 = egvl['Ws'][:, :ECH] @ jnp.abs(w1e)
    Wse2 = egvl['Ws'][:, ECH:]
    Wd2 = p['edge_act_Wd'] * w2e[None, :]

    node_w = []
    for g in (ngvl, cgvl):
        node_w += [g['Wv1'], g['Wv2'], g['Ws'][:, :VEC], g['Ws'][:, VEC:],
                   g['Wg'], r1(g['bg'])]
    edge_w = [r1(s1), Wse2, egvl['Wg'], r1(egvl['bg']), r1(w2e), Wd2,
              p['sca_W'], r1(p['sca_b']), p['e2n_W'], r1(p['e2n_b']),
              p['n2e_W'], r1(p['n2e_b']), p['edge_vnl_W'],
              mgvl['Wv1'], mgvl['Wv2'], mgvl['Ws'][:, :VEC],
              mgvl['Ws'][:, VEC:], mgvl['Wg'], r1(mgvl['bg'])]
    out_w = [r1(p['ln_sca_w']), r1(p['ln_sca_b']),
             r1(p['ln_vec_w'].T.reshape(-1)), r1(p['ln_vec_b'].T.reshape(-1)),
             p['act_vec_Wd'],
             ogvl['Wv1'], ogvl['Wv2'], ogvl['Ws'][:, :VEC], ogvl['Ws'][:, VEC:],
             ogvl['Wg'], r1(ogvl['bg'])]

    xs = jnp.pad(x_scalar, ((0, NODE_PAD - N_NODES), (0, 0)))
    xv = jnp.pad(x_vector.transpose(0, 2, 1).reshape(N_NODES, 3 * VEC),
                 ((0, NODE_PAD - N_NODES), (0, 0)))
    ncat, ccat = pl.pallas_call(
        _node_pre_body,
        grid=(NODE_PAD // NODE_BLK,),
        in_specs=[_row_spec(NODE_BLK, SCA), _row_spec(NODE_BLK, 3 * VEC)]
        + [_full_spec(w.shape) for w in node_w],
        out_specs=[_row_spec(NODE_BLK, GCAT), _row_spec(NODE_BLK, CAT)],
        out_shape=[jax.ShapeDtypeStruct((NODE_PAD, GCAT), f32),
                   jax.ShapeDtypeStruct((NODE_PAD, CAT), f32)],
    )(xs, xv, *node_w)

    row = edge_index[0]
    col = edge_index[1]
    colp = jnp.pad(col, (0, EDGE_PAD - N_EDGES))
    rowp = jnp.pad(row, (0, EDGE_PAD - N_EDGES), constant_values=N_NODES)
    gath = _sc_gather(ncat, colp.reshape(EDGE_PAD // 128, 128))

    eb = jnp.concatenate(
        [edge_vector, edge_feature,
         jnp.zeros((N_EDGES, 1), f32)], axis=1)
    eb = jnp.pad(eb, ((0, EDGE_PAD - N_EDGES), (0, 0)))
    ebt = eb.T
    msgL, msgR = pl.pallas_call(
        _edge_body,
        grid=(EDGE_PAD // EDGE_BLK,),
        in_specs=[_row_spec(EDGE_BLK, 8),
                  pl.BlockSpec((8, EDGE_BLK), lambda i: (0, i)),
                  _row_spec(EDGE_BLK, GCAT)]
        + [_full_spec(w.shape) for w in edge_w],
        out_specs=[_row_spec(EDGE_BLK, HALF)] * 2,
        out_shape=[jax.ShapeDtypeStruct((EDGE_PAD, HALF), f32)] * 2,
    )(eb, ebt, gath, *edge_w)

    zeros_h = jnp.zeros((NODE_PAD // NS, HALF), f32)
    aggL, aggR = _sc_scatter(rowp.reshape(EDGE_PAD // 128, 128),
                             msgL, msgR, zeros_h)

    outs, outv = pl.pallas_call(
        _node_out_body,
        grid=(NODE_PAD // NODE_BLK,),
        in_specs=[_row_spec(NODE_BLK, HALF)] * 2
        + [_row_spec(NODE_BLK, CAT)]
        + [_full_spec(w.shape) for w in out_w],
        out_specs=[_row_spec(NODE_BLK, SCA), _row_spec(NODE_BLK, 3 * VEC)],
        out_shape=[jax.ShapeDtypeStruct((NODE_PAD, SCA), f32),
                   jax.ShapeDtypeStruct((NODE_PAD, 3 * VEC), f32)],
    )(aggL, aggR, ccat, *out_w)

    out_sca = outs[:N_NODES]
    out_vec = outv[:N_NODES].reshape(N_NODES, 3, VEC).transpose(0, 2, 1)
    return out_sca, out_vec


# speedup vs baseline: 1.0391x; 1.0391x over previous
import functools
from math import pi as PI

import jax
import jax.numpy as jnp
import numpy as np
from jax import lax
from jax.experimental import pallas as pl
from jax.experimental.pallas import tpu as pltpu
from jax.experimental.pallas import tpu_sc as plsc

N_NODES = 10000
N_EDGES = 160000
SCA = 128
VEC = 32
ECH = 64
NET = 4
CUTOFF = 10.0
NG = ECH - NET
EPS = 1e-6

NODE_BLK = 2048
NODE_PAD = 10240
EDGE_BLK = 4096
EDGE_PAD = 163840

_F32 = jnp.float32

NC = 2
NS = 16
NW = NC * NS
CAT = SCA + 3 * VEC
GCAT = 256
HALF = 128
GCH = 256
GPW = EDGE_PAD // NW
SCH = 256
SPT = EDGE_PAD // NS


def _sc_gather(table, col2):
    mesh = plsc.VectorSubcoreMesh(core_axis_name="c", subcore_axis_name="s")

    @functools.partial(
        pl.kernel, mesh=mesh,
        out_type=jax.ShapeDtypeStruct((EDGE_PAD, GCAT), _F32),
        scratch_types=[pltpu.VMEM((8, 128), jnp.int32),
                       pltpu.VMEM((GCH, GCAT), _F32),
                       pltpu.SemaphoreType.DMA],
    )
    def k(table_h, col_h, out_h, idx_v, rows_v, sem):
        w = lax.axis_index("s") * NC + lax.axis_index("c")
        for t in range(GPW // 1024):
            r = pl.multiple_of((t * NW + w) * 8, 8)
            pltpu.sync_copy(col_h.at[pl.ds(r, 8)], idx_v)
            for u in range(1024 // GCH):
                cps = [pltpu.async_copy(
                    table_h.at[idx_v.at[u * (GCH // 128) + j]],
                    rows_v.at[pl.ds(j * 128, 128)], sem)
                    for j in range(GCH // 128)]
                for cp in cps:
                    cp.wait()
                pltpu.sync_copy(
                    rows_v,
                    out_h.at[pl.ds((t * NW + w) * 1024 + u * GCH, GCH)])

    return k(table, col2)


def _sc_scatter(row2, msg2, zeros_h):
    mesh = plsc.VectorSubcoreMesh(core_axis_name="c", subcore_axis_name="s")

    @functools.partial(
        pl.kernel, mesh=mesh,
        out_type=[jax.ShapeDtypeStruct((NODE_PAD, HALF), _F32)] * 2,
        scratch_types=[pltpu.VMEM((8, 128), jnp.int32),
                       pltpu.VMEM((2, 128, HALF), _F32),
                       pltpu.VMEM_SHARED((NODE_PAD, HALF), _F32),
                       pltpu.SemaphoreType.DMA,
                       pltpu.SemaphoreType.DMA],
    )
    def k(row_h, msg_h, z_h, outL_h, outR_h, idx_v, buf_v, acc_s, sem0, sem1):
        ci = lax.axis_index("c")
        si = lax.axis_index("s")
        zrows = NODE_PAD // NS
        z0 = pl.multiple_of(si * zrows, 8)
        pltpu.sync_copy(z_h, acc_s.at[pl.ds(z0, zrows)])
        plsc.subcore_barrier()
        sems = [sem0, sem1]
        nchunks = SPT // 128

        def chunk_base(i):
            b, j = divmod(i, 8)
            return b * NS * 1024 + si * 1024 + j * 128

        def start_load(i):
            p = i % 2
            return pltpu.async_copy(
                msg_h.at[ci, pl.ds(chunk_base(i), 128)], buf_v.at[p], sems[p])

        def load_idx(i):
            b = i // 8
            r = pl.multiple_of((b * NS + si) * 8, 8)
            pltpu.sync_copy(row_h.at[pl.ds(r, 8)], idx_v)

        load_idx(0)
        pend = [start_load(0), None]
        for i in range(nchunks):
            p = i % 2
            if i + 1 < nchunks:
                pend[1 - p] = start_load(i + 1)
            pend[p].wait()
            pltpu.sync_copy(buf_v.at[p], acc_s.at[idx_v.at[i % 8]], add=True)
            if i + 1 < nchunks and (i + 1) % 8 == 0:
                load_idx(i + 1)
        plsc.subcore_barrier()
        orows = NODE_PAD // NS
        o0 = pl.multiple_of(si * orows, 8)

        @pl.when(ci == 0)
        def _():
            pltpu.sync_copy(acc_s.at[pl.ds(o0, orows)],
                            outL_h.at[pl.ds(o0, orows)])

        @pl.when(ci == 1)
        def _():
            pltpu.sync_copy(acc_s.at[pl.ds(o0, orows)],
                            outR_h.at[pl.ds(o0, orows)])

    return k(row2, msg2, zeros_h)


def _leaky01(x):
    return jnp.where(x >= 0, x, 0.01 * x)


def _dot(a, b):
    return jax.lax.dot_general(a, b, (((1,), (1,)), ((), ())),
                               preferred_element_type=_F32)


def _gvl_full(sca, v0, v1, v2, Wv1, Wv2, Ws1, Ws2, Wg, bg):
    i0 = _dot(v0, Wv1)
    i1 = _dot(v1, Wv1)
    i2 = _dot(v2, Wv1)
    vn = jnp.sqrt(i0 * i0 + i1 * i1 + i2 * i2)
    os_ = _dot(vn, Ws1) + _dot(sca, Ws2)
    o0 = _dot(i0, Wv2)
    o1 = _dot(i1, Wv2)
    o2 = _dot(i2, Wv2)
    g = jax.nn.sigmoid(_dot(os_, Wg) + bg)
    return os_, g * o0, g * o1, g * o2


def _node_pre_body(xs_ref, xv_ref,
                   nWv1, nWv2, nWs1, nWs2, nWg, nbg,
                   cWv1, cWv2, cWs1, cWs2, cWg, cbg,
                   ncat_ref, ccat_ref):
    xs = xs_ref[...]
    xv = xv_ref[...]
    v0 = xv[:, 0 * VEC:1 * VEC]
    v1 = xv[:, 1 * VEC:2 * VEC]
    v2 = xv[:, 2 * VEC:3 * VEC]
    ns, n0, n1, n2 = _gvl_full(xs, v0, v1, v2, nWv1[...], nWv2[...],
                               nWs1[...], nWs2[...], nWg[...], nbg[...])
    cs, c0, c1, c2 = _gvl_full(xs, v0, v1, v2, cWv1[...], cWv2[...],
                               cWs1[...], cWs2[...], cWg[...], cbg[...])
    B = xs.shape[0]
    ncat_ref[...] = jnp.concatenate(
        [ns, n0, n1, n2, jnp.zeros((B, GCAT - CAT), _F32)], axis=1)
    ccat_ref[...] = jnp.concatenate([cs, c0, c1, c2], axis=1)


def _edge_body(eb_ref, ebt_ref, nc_ref,
               s1, Wse2, Wge, bge, w2e, Wd2,
               scaW, scab, e2nW, e2nb, n2eW, n2eb, evnlW,
               mWv1, mWv2, mWs1, mWs2, mWg, mbg,
               msg_ref):
    eb = eb_ref[...]
    B = eb.shape[0]
    ebt = ebt_ref[...]
    e0t = ebt[0:1, :]
    e1t = ebt[1:2, :]
    e2t = ebt[2:3, :]
    distt = jnp.sqrt(e0t * e0t + e1t * e1t + e2t * e2t)
    invt = 1.0 / (distt + 1e-7)
    unt = distt * invt
    Cct = 0.5 * (jnp.cos(distt * (PI / CUTOFF)) + 1.0)
    Cct = Cct * (distt <= CUTOFF).astype(_F32)

    def tb(x):
        return x.reshape(B, 1)

    dist = tb(distt)
    un = tb(unt)
    un2 = tb(unt * unt)
    u0 = tb(e0t * invt)
    u1 = tb(e1t * invt)
    u2 = tb(e2t * invt)
    Cc = tb(Cct)

    colf = jax.lax.broadcasted_iota(jnp.int32, (B, ECH), 1)
    step = CUTOFF / (NG - 1)
    off = colf.astype(_F32) * step
    coeff = -0.5 / (step * step)
    gs = jnp.exp(coeff * (dist - off) ** 2)
    feat = jnp.where(colf < NG, gs, 0.0)
    for j in range(NET):
        feat = jnp.where(colf == NG + j, eb[:, 3 + j:4 + j], feat)

    ose = un * s1[...] + _dot(feat, Wse2[...])
    ge = jax.nn.sigmoid(_dot(ose, Wge[...]) + bge[...])
    a = ge * w2e[...]
    h = _dot(ge, Wd2[...])
    dvd = a * h * un2
    dsq = h * h * un2
    mask = (dvd >= 0).astype(_F32)
    ecoef = 0.2 * a + 0.8 * (mask * a + (1.0 - mask) * (a - dvd / (dsq + EPS) * h))
    esca = _leaky01(ose)

    nc = nc_ref[...]
    nsj = nc[:, :SCA]
    nv0 = nc[:, SCA + 0 * VEC:SCA + 1 * VEC]
    nv1 = nc[:, SCA + 1 * VEC:SCA + 2 * VEC]
    nv2 = nc[:, SCA + 2 * VEC:SCA + 3 * VEC]

    ysca = nsj * (_dot(esca, scaW[...]) + scab[...])
    A = _dot(esca, e2nW[...]) + e2nb[...]
    Bc = _dot(nsj, n2eW[...]) + n2eb[...]
    q = _dot(ecoef, evnlW[...])
    Bq = Bc * q
    yv0 = A * nv0 + Bq * u0
    yv1 = A * nv1 + Bq * u1
    yv2 = A * nv2 + Bq * u2

    i0 = _dot(yv0, mWv1[...])
    i1 = _dot(yv1, mWv1[...])
    i2 = _dot(yv2, mWv1[...])
    vn = jnp.sqrt(i0 * i0 + i1 * i1 + i2 * i2)
    osm = _dot(vn, mWs1[...]) + _dot(ysca, mWs2[...])
    gm = jax.nn.sigmoid(_dot(osm, mWg[...]) + mbg[...])
    o0 = gm * _dot(i0, mWv2[...])
    o1 = gm * _dot(i1, mWv2[...])
    o2 = gm * _dot(i2, mWv2[...])

    msg_ref[0] = osm * Cc
    msg_ref[1] = jnp.concatenate(
        [o0 * Cc, o1 * Cc, o2 * Cc,
         jnp.zeros((B, HALF - 3 * VEC), _F32)], axis=1)


def _node_out_body(aggL_ref, aggR_ref, ccat_ref,
                   lnsw, lnsb, lnvw, lnvb, Wdv,
                   oWv1, oWv2, oWs1, oWs2, oWg, obg,
                   os_ref, ov_ref):
    agL = aggL_ref[...]
    agR = aggR_ref[...]
    cc = ccat_ref[...]
    s = agL + cc[:, :SCA]
    vall = agR[:, :3 * VEC] + cc[:, SCA:]

    m = jnp.mean(s, axis=1, keepdims=True)
    var = jnp.mean((s - m) ** 2, axis=1, keepdims=True)
    sN = (s - m) / jnp.sqrt(var + 1e-5) * lnsw[...] + lnsb[...]

    m2 = jnp.mean(vall, axis=1, keepdims=True)
    var2 = jnp.mean((vall - m2) ** 2, axis=1, keepdims=True)
    vN = (vall - m2) / jnp.sqrt(var2 + 1e-5) * lnvw[...] + lnvb[...]
    v0 = vN[:, 0 * VEC:1 * VEC]
    v1 = vN[:, 1 * VEC:2 * VEC]
    v2 = vN[:, 2 * VEC:3 * VEC]

    sA = _leaky01(sN)
    d0 = _dot(v0, Wdv[...])
    d1 = _dot(v1, Wdv[...])
    d2 = _dot(v2, Wdv[...])
    dvd = v0 * d0 + v1 * d1 + v2 * d2
    dsq = d0 * d0 + d1 * d1 + d2 * d2
    mask = (dvd >= 0).astype(_F32)
    co = dvd / (dsq + EPS)
    w0 = 0.2 * v0 + 0.8 * (mask * v0 + (1.0 - mask) * (v0 - co * d0))
    w1 = 0.2 * v1 + 0.8 * (mask * v1 + (1.0 - mask) * (v1 - co * d1))
    w2 = 0.2 * v2 + 0.8 * (mask * v2 + (1.0 - mask) * (v2 - co * d2))

    fs, f0, f1, f2 = _gvl_full(sA, w0, w1, w2, oWv1[...], oWv2[...],
                               oWs1[...], oWs2[...], oWg[...], obg[...])
    os_ref[...] = fs
    ov_ref[...] = jnp.concatenate([f0, f1, f2], axis=1)


def _full_spec(shape):
    nd = len(shape)
    return pl.BlockSpec(shape, lambda i, _n=nd: (0,) * _n)


def _row_spec(blk, ncols):
    return pl.BlockSpec((blk, ncols), lambda i: (i, 0))


def kernel(x_scalar, x_vector, edge_feature, edge_vector, params, edge_index):
    p = params
    f32 = _F32

    def r1(x):
        return x.reshape(1, -1).astype(f32)

    ngvl, cgvl, egvl, mgvl, ogvl = (p['node_gvl'], p['centroid_gvl'],
                                    p['edge_gvl'], p['mm_out_gvl'],
                                    p['out_gvl'])
    w1e = egvl['Wv1'] @ p['ee_w']
    w2e = egvl['Wv2'] @ w1e
    s1 = egvl['Ws'][:, :ECH] @ jnp.abs(w1e)
    Wse2 = egvl['Ws'][:, ECH:]
    Wd2 = p['edge_act_Wd'] * w2e[None, :]

    node_w = []
    for g in (ngvl, cgvl):
        node_w += [g['Wv1'], g['Wv2'], g['Ws'][:, :VEC], g['Ws'][:, VEC:],
                   g['Wg'], r1(g['bg'])]
    edge_w = [r1(s1), Wse2, egvl['Wg'], r1(egvl['bg']), r1(w2e), Wd2,
              p['sca_W'], r1(p['sca_b']), p['e2n_W'], r1(p['e2n_b']),
              p['n2e_W'], r1(p['n2e_b']), p['edge_vnl_W'],
              mgvl['Wv1'], mgvl['Wv2'], mgvl['Ws'][:, :VEC],
              mgvl['Ws'][:, VEC:], mgvl['Wg'], r1(mgvl['bg'])]
    out_w = [r1(p['ln_sca_w']), r1(p['ln_sca_b']),
             r1(p['ln_vec_w'].T.reshape(-1)), r1(p['ln_vec_b'].T.reshape(-1)),
             p['act_vec_Wd'],
             ogvl['Wv1'], ogvl['Wv2'], ogvl['Ws'][:, :VEC], ogvl['Ws'][:, VEC:],
             ogvl['Wg'], r1(ogvl['bg'])]

    xs = jnp.pad(x_scalar, ((0, NODE_PAD - N_NODES), (0, 0)))
    xv = jnp.pad(x_vector.transpose(0, 2, 1).reshape(N_NODES, 3 * VEC),
                 ((0, NODE_PAD - N_NODES), (0, 0)))
    ncat, ccat = pl.pallas_call(
        _node_pre_body,
        grid=(NODE_PAD // NODE_BLK,),
        in_specs=[_row_spec(NODE_BLK, SCA), _row_spec(NODE_BLK, 3 * VEC)]
        + [_full_spec(w.shape) for w in node_w],
        out_specs=[_row_spec(NODE_BLK, GCAT), _row_spec(NODE_BLK, CAT)],
        out_shape=[jax.ShapeDtypeStruct((NODE_PAD, GCAT), f32),
                   jax.ShapeDtypeStruct((NODE_PAD, CAT), f32)],
    )(xs, xv, *node_w)

    row = edge_index[0]
    col = edge_index[1]
    colp = jnp.pad(col, (0, EDGE_PAD - N_EDGES))
    rowp = jnp.pad(row, (0, EDGE_PAD - N_EDGES), constant_values=N_NODES)
    gath = _sc_gather(ncat, colp.reshape(EDGE_PAD // 128, 128))

    eb = jnp.concatenate(
        [edge_vector, edge_feature,
         jnp.zeros((N_EDGES, 1), f32)], axis=1)
    eb = jnp.pad(eb, ((0, EDGE_PAD - N_EDGES), (0, 0)))
    ebt = eb.T
    msg2 = pl.pallas_call(
        _edge_body,
        grid=(EDGE_PAD // EDGE_BLK,),
        in_specs=[_row_spec(EDGE_BLK, 8),
                  pl.BlockSpec((8, EDGE_BLK), lambda i: (0, i)),
                  _row_spec(EDGE_BLK, GCAT)]
        + [_full_spec(w.shape) for w in edge_w],
        out_specs=[pl.BlockSpec((2, EDGE_BLK, HALF), lambda i: (0, i, 0))],
        out_shape=[jax.ShapeDtypeStruct((2, EDGE_PAD, HALF), f32)],
    )(eb, ebt, gath, *edge_w)[0]

    zeros_h = jnp.zeros((NODE_PAD // NS, HALF), f32)
    aggL, aggR = _sc_scatter(rowp.reshape(EDGE_PAD // 128, 128),
                             msg2, zeros_h)

    outs, outv = pl.pallas_call(
        _node_out_body,
        grid=(NODE_PAD // NODE_BLK,),
        in_specs=[_row_spec(NODE_BLK, HALF)] * 2
        + [_row_spec(NODE_BLK, CAT)]
        + [_full_spec(w.shape) for w in out_w],
        out_specs=[_row_spec(NODE_BLK, SCA), _row_spec(NODE_BLK, 3 * VEC)],
        out_shape=[jax.ShapeDtypeStruct((NODE_PAD, SCA), f32),
                   jax.ShapeDtypeStruct((NODE_PAD, 3 * VEC), f32)],
    )(aggL, aggR, ccat, *out_w)

    out_sca = outs[:N_NODES]
    out_vec = outv[:N_NODES].reshape(N_NODES, 3, VEC).transpose(0, 2, 1)
    return out_sca, out_vec
